# Initial kernel scaffold; baseline (speedup 1.0000x reference)
import jax
import jax.numpy as jnp
from jax.experimental import pallas as pl


def kernel(input_ids, embedding_table):
    raise NotImplementedError("write your pallas kernel here")


# speedup vs baseline: 1.8581x; 1.8581x over previous
import jax
import jax.numpy as jnp
from jax import lax
from jax.experimental import pallas as pl
from jax.experimental.pallas import tpu as pltpu
from jax.experimental.pallas import tpu_sc as plsc

_EMBED_DIM = 64
_NC = 2
_NS = 16
_NW = _NC * _NS
_SUB = 128
_SUBS_PER_CHUNK = 8
_CHUNK = _SUB * _SUBS_PER_CHUNK


def _gather_body(table_hbm, idx_hbm, out_hbm, idx_v, rows_v, sem):
    wid = lax.axis_index("s") * _NC + lax.axis_index("c")
    n_chunks = out_hbm.shape[0] // (_NW * _CHUNK)
    base_row = wid * n_chunks * _CHUNK
    base_idx_row = wid * n_chunks * _SUBS_PER_CHUNK

    def chunk_body(i, carry):
        off = pl.multiple_of(base_row + i * _CHUNK, _CHUNK)
        idx_off = base_idx_row + i * _SUBS_PER_CHUNK
        pltpu.sync_copy(idx_hbm.at[pl.ds(idx_off, _SUBS_PER_CHUNK)], idx_v)
        copies = [
            pltpu.async_copy(
                table_hbm.at[idx_v.at[j]],
                rows_v.at[pl.ds(j * _SUB, _SUB)],
                sem,
            )
            for j in range(_SUBS_PER_CHUNK)
        ]
        for c in copies:
            c.wait()
        pltpu.sync_copy(rows_v, out_hbm.at[pl.ds(off, _CHUNK)])
        return carry

    lax.fori_loop(0, n_chunks, chunk_body, 0)


@jax.jit
def _embedding_gather(table, idx2d):
    total = idx2d.shape[0] * idx2d.shape[1]
    mesh = plsc.VectorSubcoreMesh(core_axis_name="c", subcore_axis_name="s")
    f = pl.kernel(
        _gather_body,
        mesh=mesh,
        out_type=jax.ShapeDtypeStruct((total, _EMBED_DIM), jnp.float32),
        scratch_types=[
            pltpu.VMEM((_SUBS_PER_CHUNK, _SUB), jnp.int32),
            pltpu.VMEM((_CHUNK, _EMBED_DIM), jnp.float32),
            pltpu.SemaphoreType.DMA,
        ],
        compiler_params=pltpu.CompilerParams(use_tc_tiling_on_sc=False),
    )
    return f(table, idx2d)


def kernel(input_ids, embedding_table):
    flat = input_ids.reshape(-1, _SUB)
    out = _embedding_gather(embedding_table, flat)
    return out.reshape(input_ids.shape + (_EMBED_DIM,))


# speedup vs baseline: 1.8664x; 1.0044x over previous
import jax
import jax.numpy as jnp
from jax import lax
from jax.experimental import pallas as pl
from jax.experimental.pallas import tpu as pltpu
from jax.experimental.pallas import tpu_sc as plsc

_EMBED_DIM = 64
_NC = 2
_NS = 16
_NW = _NC * _NS
_SUB = 128
_SUBS_PER_CHUNK = 2
_CHUNK = _SUB * _SUBS_PER_CHUNK
_NBUF = 5


def _gather_body(table_hbm, idx_hbm, out_hbm, idx_v, rows_v, gsem, wsem):
    wid = lax.axis_index("s") * _NC + lax.axis_index("c")
    total = out_hbm.shape[0]
    n_chunks = total // (_NW * _CHUNK)
    n_idx_rows = n_chunks * _SUBS_PER_CHUNK
    n_groups = n_chunks // _NBUF
    base_row = wid * n_chunks * _CHUNK

    pltpu.sync_copy(idx_hbm.at[pl.ds(wid * n_idx_rows, n_idx_rows)], idx_v)

    def group_body(g, carry):
        c0 = g * _NBUF
        gds = []
        for b in range(_NBUF):
            @pl.when(g > 0)
            def _():
                pltpu.make_async_copy(
                    rows_v.at[b], out_hbm.at[pl.ds(0, _CHUNK)], wsem.at[b]
                ).wait()
            slot = []
            for j in range(_SUBS_PER_CHUNK):
                slot.append(pltpu.async_copy(
                    table_hbm.at[idx_v.at[(c0 + b) * _SUBS_PER_CHUNK + j]],
                    rows_v.at[b].at[pl.ds(j * _SUB, _SUB)],
                    gsem.at[b],
                ))
            gds.append(slot)
        for b in range(_NBUF):
            for d in gds[b]:
                d.wait()
            off = pl.multiple_of(base_row + (c0 + b) * _CHUNK, _CHUNK)
            pltpu.async_copy(rows_v.at[b], out_hbm.at[pl.ds(off, _CHUNK)],
                             wsem.at[b])
        return carry

    lax.fori_loop(0, n_groups, group_body, 0)

    for b in range(_NBUF):
        pltpu.make_async_copy(
            rows_v.at[b], out_hbm.at[pl.ds(0, _CHUNK)], wsem.at[b]
        ).wait()


@jax.jit
def _embedding_gather(table, idx2d):
    total = idx2d.shape[0] * idx2d.shape[1]
    n_idx_rows = total // (_NW * _SUB)
    mesh = plsc.VectorSubcoreMesh(core_axis_name="c", subcore_axis_name="s")
    f = pl.kernel(
        _gather_body,
        mesh=mesh,
        out_type=jax.ShapeDtypeStruct((total, _EMBED_DIM), jnp.float32),
        scratch_types=[
            pltpu.VMEM((n_idx_rows, _SUB), jnp.int32),
            pltpu.VMEM((_NBUF, _CHUNK, _EMBED_DIM), jnp.float32),
            pltpu.SemaphoreType.DMA((_NBUF,)),
            pltpu.SemaphoreType.DMA((_NBUF,)),
        ],
        compiler_params=pltpu.CompilerParams(use_tc_tiling_on_sc=False),
    )
    return f(table, idx2d)


def kernel(input_ids, embedding_table):
    flat = input_ids.reshape(-1, _SUB)
    out = _embedding_gather(embedding_table, flat)
    return out.reshape(input_ids.shape + (_EMBED_DIM,))
